# Initial kernel scaffold; baseline (speedup 1.0000x reference)
import jax
import jax.numpy as jnp
from jax.experimental import pallas as pl


def kernel(x, edge_index, edge_weight, W1, b1, W2, b2):
    raise NotImplementedError("write your pallas kernel here")


# speedup vs baseline: 4.7351x; 4.7351x over previous
import functools

import jax
import jax.numpy as jnp
from jax import lax
from jax.experimental import pallas as pl
from jax.experimental.pallas import tpu as pltpu
from jax.experimental.pallas import tpu_sc as plsc

N = 10000
E = 320000
CH = 128
NCORE = 2
NSUB = 16
NWORK = NCORE * NSUB
ROWS_PER_SUB = 640
NPAD = ROWS_PER_SUB * NSUB
NCHUNK = E // CH


def _make_spmm(D):
    mesh = plsc.VectorSubcoreMesh(core_axis_name="c", subcore_axis_name="s",
                                  num_cores=NCORE, num_subcores=NSUB)
    nvec = D // 16

    @functools.partial(
        pl.kernel,
        out_type=jax.ShapeDtypeStruct((NCORE, NPAD, D), jnp.float32),
        mesh=mesh,
        scratch_types=[
            pltpu.VMEM((CH,), jnp.int32),
            pltpu.VMEM((CH,), jnp.int32),
            pltpu.VMEM((CH,), jnp.float32),
            pltpu.VMEM((CH, D), jnp.float32),
            pltpu.VMEM((ROWS_PER_SUB, D), jnp.float32),
            pltpu.VMEM_SHARED((NPAD, D), jnp.float32),
            pltpu.SemaphoreType.DMA,
        ],
        compiler_params=pltpu.CompilerParams(use_tc_tiling_on_sc=False),
    )
    def spmm(table_ref, src_ref, dst_ref, w_ref, out_ref,
             sidx, didx, wbuf, rows, zbuf, acc, sem):
        c = lax.axis_index("c")
        s = lax.axis_index("s")
        wid = c * NSUB + s

        zv = jnp.zeros((16,), jnp.float32)

        def zbody(i, carry):
            for jj in range(nvec):
                zbuf[i, pl.ds(jj * 16, 16)] = zv
            return carry

        lax.fori_loop(0, ROWS_PER_SUB, zbody, 0)
        pltpu.sync_copy(zbuf, acc.at[pl.ds(s * ROWS_PER_SUB, ROWS_PER_SUB)])
        plsc.subcore_barrier()

        nch = (NCHUNK - 1 - wid) // NWORK + 1

        def chunk_body(t, carry):
            base = (wid + t * NWORK) * CH
            pltpu.sync_copy(src_ref.at[pl.ds(base, CH)], sidx)
            pltpu.sync_copy(dst_ref.at[pl.ds(base, CH)], didx)
            pltpu.sync_copy(w_ref.at[pl.ds(base, CH)], wbuf)
            pltpu.async_copy(table_ref.at[sidx], rows, sem).wait()

            def sbody(g, icarry):
                wvec = wbuf[pl.ds(g * 16, 16)]
                for k in range(16):
                    wv = wvec[k]
                    i = g * 16 + k
                    for jj in range(nvec):
                        sl = pl.ds(jj * 16, 16)
                        rows[i, sl] = rows[i, sl] * wv
                return icarry

            lax.fori_loop(0, CH // 16, sbody, 0)
            pltpu.sync_copy(rows, acc.at[didx], add=True)
            return carry

        lax.fori_loop(0, nch, chunk_body, 0)
        plsc.subcore_barrier()
        pltpu.sync_copy(acc.at[pl.ds(s * ROWS_PER_SUB, ROWS_PER_SUB)],
                        out_ref.at[c, pl.ds(s * ROWS_PER_SUB, ROWS_PER_SUB)])

    return spmm


_SPMM_CACHE = {}


def _get_spmm(D):
    if D not in _SPMM_CACHE:
        _SPMM_CACHE[D] = _make_spmm(D)
    return _SPMM_CACHE[D]


def _mm1_body(x_ref, w_ref, o_ref):
    o_ref[...] = jnp.dot(x_ref[...], w_ref[...],
                         preferred_element_type=jnp.float32)


def _stage2_body(p_ref, b1_ref, w2_ref, o_ref):
    h = jnp.maximum(p_ref[0] + p_ref[1] + b1_ref[...], 0.0)
    o_ref[...] = jnp.dot(h, w2_ref[...], preferred_element_type=jnp.float32)


def _stage3_body(q_ref, b2_ref, lo_ref, lp_ref):
    lg = q_ref[0] + q_ref[1] + b2_ref[...]
    m = jnp.max(lg, axis=1, keepdims=True)
    se = jnp.sum(jnp.exp(lg - m), axis=1, keepdims=True)
    lo_ref[...] = lg
    lp_ref[...] = lg - m - jnp.log(se)


def kernel(x, edge_index, edge_weight, W1, b1, W2, b2):
    src = edge_index[1].astype(jnp.int32)
    dst = edge_index[0].astype(jnp.int32)
    ew = edge_weight.astype(jnp.float32)

    support1 = pl.pallas_call(
        _mm1_body,
        out_shape=jax.ShapeDtypeStruct((N, 64), jnp.float32),
    )(x, W1)

    p = _get_spmm(64)(support1, src, dst, ew)[:, :N]

    support2 = pl.pallas_call(
        _stage2_body,
        out_shape=jax.ShapeDtypeStruct((N, 16), jnp.float32),
    )(p, b1.reshape(1, 64), W2)

    q = _get_spmm(16)(support2, src, dst, ew)[:, :N]

    logits, logp = pl.pallas_call(
        _stage3_body,
        out_shape=[jax.ShapeDtypeStruct((N, 16), jnp.float32),
                   jax.ShapeDtypeStruct((N, 16), jnp.float32)],
    )(q, b2.reshape(1, 16))

    return (logits, logp)


# speedup vs baseline: 8.6682x; 1.8306x over previous
import functools

import jax
import jax.numpy as jnp
from jax import lax
from jax.experimental import pallas as pl
from jax.experimental.pallas import tpu as pltpu
from jax.experimental.pallas import tpu_sc as plsc

N = 10000
E = 320000
CH = 128
NCORE = 2
NSUB = 16
NWORK = NCORE * NSUB
ROWS_PER_SUB = 640
NPAD = ROWS_PER_SUB * NSUB
NCHUNK = E // CH


CHW_BASE = NCHUNK // NWORK
CHW_EXTRA = NCHUNK - CHW_BASE * NWORK
EBASE = CHW_BASE * CH
EBUF = (CHW_BASE + 1) * CH


def _make_spmm(D):
    mesh = plsc.VectorSubcoreMesh(core_axis_name="c", subcore_axis_name="s",
                                  num_cores=NCORE, num_subcores=NSUB)
    nvec = D // 16

    @functools.partial(
        pl.kernel,
        out_type=jax.ShapeDtypeStruct((NCORE, NPAD, D), jnp.float32),
        mesh=mesh,
        scratch_types=[
            pltpu.VMEM((EBUF,), jnp.int32),
            pltpu.VMEM((EBUF,), jnp.int32),
            pltpu.VMEM((EBUF,), jnp.float32),
            pltpu.VMEM((2, CH, D), jnp.float32),
            pltpu.VMEM((ROWS_PER_SUB, D), jnp.float32),
            pltpu.VMEM_SHARED((NPAD, D), jnp.float32),
            pltpu.SemaphoreType.DMA,
            pltpu.SemaphoreType.DMA,
        ],
        compiler_params=pltpu.CompilerParams(use_tc_tiling_on_sc=False),
    )
    def spmm(table_ref, src_ref, dst_ref, w_ref, out_ref,
             sidx, didx, wbuf, rows, zbuf, acc, gsem, ssem):
        c = lax.axis_index("c")
        s = lax.axis_index("s")
        wid = c * NSUB + s
        nch = CHW_BASE + jnp.where(wid < CHW_EXTRA, 1, 0)
        e0 = (wid * CHW_BASE + jnp.minimum(wid, CHW_EXTRA)) * CH

        pre1 = pltpu.async_copy(src_ref.at[pl.ds(e0, EBASE)],
                                sidx.at[pl.ds(0, EBASE)], gsem)
        pre2 = pltpu.async_copy(dst_ref.at[pl.ds(e0, EBASE)],
                                didx.at[pl.ds(0, EBASE)], gsem)
        pre3 = pltpu.async_copy(w_ref.at[pl.ds(e0, EBASE)],
                                wbuf.at[pl.ds(0, EBASE)], gsem)

        zv = jnp.zeros((16,), jnp.float32)

        def zbody(i, carry):
            for jj in range(nvec):
                zbuf[i, pl.ds(jj * 16, 16)] = zv
            return carry

        lax.fori_loop(0, ROWS_PER_SUB, zbody, 0)
        pltpu.sync_copy(zbuf, acc.at[pl.ds(s * ROWS_PER_SUB, ROWS_PER_SUB)])

        @pl.when(wid < CHW_EXTRA)
        def _():
            pltpu.sync_copy(src_ref.at[pl.ds(e0 + EBASE, CH)],
                            sidx.at[pl.ds(EBASE, CH)])
            pltpu.sync_copy(dst_ref.at[pl.ds(e0 + EBASE, CH)],
                            didx.at[pl.ds(EBASE, CH)])
            pltpu.sync_copy(w_ref.at[pl.ds(e0 + EBASE, CH)],
                            wbuf.at[pl.ds(EBASE, CH)])

        pre1.wait()
        pre2.wait()
        pre3.wait()
        plsc.subcore_barrier()

        pltpu.async_copy(table_ref.at[sidx.at[pl.ds(0, CH)]], rows.at[0],
                         gsem)

        def chunk_body(t, carry):
            b = lax.rem(t, 2)
            pltpu.make_async_copy(table_ref.at[sidx.at[pl.ds(0, CH)]],
                                  rows.at[b], gsem).wait()

            @pl.when(t >= 1)
            def _():
                pltpu.make_async_copy(table_ref.at[sidx.at[pl.ds(0, CH)]],
                                      rows.at[1 - b], ssem).wait()

            @pl.when(t + 1 < nch)
            def _():
                pltpu.async_copy(
                    table_ref.at[sidx.at[pl.ds((t + 1) * CH, CH)]],
                    rows.at[1 - b], gsem)

            def sbody(g, icarry):
                wvec = wbuf[pl.ds(t * CH + g * 16, 16)]
                for k in range(16):
                    wv = wvec[k]
                    i = g * 16 + k
                    for jj in range(nvec):
                        sl = pl.ds(jj * 16, 16)
                        rows[b, i, sl] = rows[b, i, sl] * wv
                return icarry

            lax.fori_loop(0, CH // 16, sbody, 0)
            pltpu.async_copy(rows.at[b], acc.at[didx.at[pl.ds(t * CH, CH)]],
                             ssem, add=True)
            return carry

        lax.fori_loop(0, nch, chunk_body, 0)
        pltpu.make_async_copy(table_ref.at[sidx.at[pl.ds(0, CH)]],
                              rows.at[0], ssem).wait()
        plsc.subcore_barrier()
        pltpu.sync_copy(acc.at[pl.ds(s * ROWS_PER_SUB, ROWS_PER_SUB)],
                        out_ref.at[c, pl.ds(s * ROWS_PER_SUB, ROWS_PER_SUB)])

    return spmm


_SPMM_CACHE = {}


def _get_spmm(D):
    if D not in _SPMM_CACHE:
        _SPMM_CACHE[D] = _make_spmm(D)
    return _SPMM_CACHE[D]


def _mm1_body(x_ref, w_ref, o_ref):
    o_ref[...] = jnp.dot(x_ref[...], w_ref[...],
                         preferred_element_type=jnp.float32)


def _stage2_body(p_ref, b1_ref, w2_ref, o_ref):
    h = jnp.maximum(p_ref[0] + p_ref[1] + b1_ref[...], 0.0)
    o_ref[...] = jnp.dot(h, w2_ref[...], preferred_element_type=jnp.float32)


def _stage3_body(q_ref, b2_ref, lo_ref, lp_ref):
    lg = q_ref[0] + q_ref[1] + b2_ref[...]
    m = jnp.max(lg, axis=1, keepdims=True)
    se = jnp.sum(jnp.exp(lg - m), axis=1, keepdims=True)
    lo_ref[...] = lg
    lp_ref[...] = lg - m - jnp.log(se)


def kernel(x, edge_index, edge_weight, W1, b1, W2, b2):
    src = edge_index[1].astype(jnp.int32)
    dst = edge_index[0].astype(jnp.int32)
    ew = edge_weight.astype(jnp.float32)

    support1 = pl.pallas_call(
        _mm1_body,
        out_shape=jax.ShapeDtypeStruct((N, 64), jnp.float32),
    )(x, W1)

    p = _get_spmm(64)(support1, src, dst, ew)[:, :N]

    support2 = pl.pallas_call(
        _stage2_body,
        out_shape=jax.ShapeDtypeStruct((N, 16), jnp.float32),
    )(p, b1.reshape(1, 64), W2)

    q = _get_spmm(16)(support2, src, dst, ew)[:, :N]

    logits, logp = pl.pallas_call(
        _stage3_body,
        out_shape=[jax.ShapeDtypeStruct((N, 16), jnp.float32),
                   jax.ShapeDtypeStruct((N, 16), jnp.float32)],
    )(q, b2.reshape(1, 16))

    return (logits, logp)


# speedup vs baseline: 9.6640x; 1.1149x over previous
import functools

import jax
import jax.numpy as jnp
from jax import lax
from jax.experimental import pallas as pl
from jax.experimental.pallas import tpu as pltpu
from jax.experimental.pallas import tpu_sc as plsc

N = 10000
E = 320000
CH = 128
NCORE = 2
NSUB = 16
NWORK = NCORE * NSUB
ROWS_PER_SUB = 640
ZROWS = 128
NPAD = ROWS_PER_SUB * NSUB
NCHUNK = E // CH


CHW_BASE = NCHUNK // NWORK
CHW_EXTRA = NCHUNK - CHW_BASE * NWORK
EBASE = CHW_BASE * CH
EBUF = (CHW_BASE + 1) * CH


def _make_spmm(D):
    mesh = plsc.VectorSubcoreMesh(core_axis_name="c", subcore_axis_name="s",
                                  num_cores=NCORE, num_subcores=NSUB)
    nvec = D // 16

    @functools.partial(
        pl.kernel,
        out_type=jax.ShapeDtypeStruct((NCORE, NPAD, D), jnp.float32),
        mesh=mesh,
        scratch_types=[
            pltpu.VMEM((EBUF,), jnp.int32),
            pltpu.VMEM((EBUF,), jnp.int32),
            pltpu.VMEM((EBUF,), jnp.float32),
            pltpu.VMEM((4, CH, D), jnp.float32),
            pltpu.VMEM((ZROWS, D), jnp.float32),
            pltpu.VMEM_SHARED((NPAD, D), jnp.float32),
            pltpu.SemaphoreType.DMA,
            pltpu.SemaphoreType.DMA,
        ],
        compiler_params=pltpu.CompilerParams(use_tc_tiling_on_sc=False),
    )
    def spmm(table_ref, src_ref, dst_ref, w_ref, out_ref,
             sidx, didx, wbuf, rows, zbuf, acc, gsem, ssem):
        c = lax.axis_index("c")
        s = lax.axis_index("s")
        wid = c * NSUB + s
        nch = CHW_BASE + jnp.where(wid < CHW_EXTRA, 1, 0)
        e0 = (wid * CHW_BASE + jnp.minimum(wid, CHW_EXTRA)) * CH

        pre1 = pltpu.async_copy(src_ref.at[pl.ds(e0, EBASE)],
                                sidx.at[pl.ds(0, EBASE)], gsem)
        pre2 = pltpu.async_copy(dst_ref.at[pl.ds(e0, EBASE)],
                                didx.at[pl.ds(0, EBASE)], gsem)
        pre3 = pltpu.async_copy(w_ref.at[pl.ds(e0, EBASE)],
                                wbuf.at[pl.ds(0, EBASE)], gsem)

        zv = jnp.zeros((16,), jnp.float32)

        def zbody(i, carry):
            for jj in range(nvec):
                zbuf[i, pl.ds(jj * 16, 16)] = zv
            return carry

        lax.fori_loop(0, ZROWS, zbody, 0)
        for k in range(ROWS_PER_SUB // ZROWS):
            pltpu.sync_copy(zbuf,
                            acc.at[pl.ds(s * ROWS_PER_SUB + k * ZROWS, ZROWS)])

        @pl.when(wid < CHW_EXTRA)
        def _():
            pltpu.sync_copy(src_ref.at[pl.ds(e0 + EBASE, CH)],
                            sidx.at[pl.ds(EBASE, CH)])
            pltpu.sync_copy(dst_ref.at[pl.ds(e0 + EBASE, CH)],
                            didx.at[pl.ds(EBASE, CH)])
            pltpu.sync_copy(w_ref.at[pl.ds(e0 + EBASE, CH)],
                            wbuf.at[pl.ds(EBASE, CH)])

        pre1.wait()
        pre2.wait()
        pre3.wait()
        plsc.subcore_barrier()

        pltpu.async_copy(table_ref.at[sidx.at[pl.ds(0, CH)]], rows.at[0],
                         gsem)

        @pl.when(1 < nch)
        def _():
            pltpu.async_copy(table_ref.at[sidx.at[pl.ds(CH, CH)]],
                             rows.at[1], gsem)

        def chunk_body(t, carry):
            b = lax.rem(t, 4)
            pltpu.make_async_copy(table_ref.at[sidx.at[pl.ds(0, CH)]],
                                  rows.at[b], gsem).wait()

            @pl.when(t >= 2)
            def _():
                pltpu.make_async_copy(table_ref.at[sidx.at[pl.ds(0, CH)]],
                                      rows.at[b], ssem).wait()

            @pl.when(t + 2 < nch)
            def _():
                pltpu.async_copy(
                    table_ref.at[sidx.at[pl.ds((t + 2) * CH, CH)]],
                    rows.at[lax.rem(t + 2, 4)], gsem)

            def sbody(g, icarry):
                wvec = wbuf[pl.ds(t * CH + g * 16, 16)]
                for k in range(16):
                    wv = wvec[k]
                    i = g * 16 + k
                    for jj in range(nvec):
                        sl = pl.ds(jj * 16, 16)
                        rows[b, i, sl] = rows[b, i, sl] * wv
                return icarry

            lax.fori_loop(0, CH // 16, sbody, 0)
            pltpu.async_copy(rows.at[b], acc.at[didx.at[pl.ds(t * CH, CH)]],
                             ssem, add=True)
            return carry

        lax.fori_loop(0, nch, chunk_body, 0)
        pltpu.make_async_copy(table_ref.at[sidx.at[pl.ds(0, CH)]],
                              rows.at[0], ssem).wait()
        pltpu.make_async_copy(table_ref.at[sidx.at[pl.ds(0, CH)]],
                              rows.at[0], ssem).wait()
        plsc.subcore_barrier()
        pltpu.sync_copy(acc.at[pl.ds(s * ROWS_PER_SUB, ROWS_PER_SUB)],
                        out_ref.at[c, pl.ds(s * ROWS_PER_SUB, ROWS_PER_SUB)])

    return spmm


_SPMM_CACHE = {}


def _get_spmm(D):
    if D not in _SPMM_CACHE:
        _SPMM_CACHE[D] = _make_spmm(D)
    return _SPMM_CACHE[D]


def _mm1_body(x_ref, w_ref, o_ref):
    o_ref[...] = jnp.dot(x_ref[...], w_ref[...],
                         preferred_element_type=jnp.float32)


def _stage2_body(p_ref, b1_ref, w2_ref, o_ref):
    h = jnp.maximum(p_ref[0] + p_ref[1] + b1_ref[...], 0.0)
    o_ref[...] = jnp.dot(h, w2_ref[...], preferred_element_type=jnp.float32)


def _stage3_body(q_ref, b2_ref, lo_ref, lp_ref):
    lg = q_ref[0] + q_ref[1] + b2_ref[...]
    m = jnp.max(lg, axis=1, keepdims=True)
    se = jnp.sum(jnp.exp(lg - m), axis=1, keepdims=True)
    lo_ref[...] = lg
    lp_ref[...] = lg - m - jnp.log(se)


def kernel(x, edge_index, edge_weight, W1, b1, W2, b2):
    src = edge_index[1].astype(jnp.int32)
    dst = edge_index[0].astype(jnp.int32)
    ew = edge_weight.astype(jnp.float32)

    support1 = pl.pallas_call(
        _mm1_body,
        out_shape=jax.ShapeDtypeStruct((N, 64), jnp.float32),
    )(x, W1)

    p = _get_spmm(64)(support1, src, dst, ew)[:, :N]

    support2 = pl.pallas_call(
        _stage2_body,
        out_shape=jax.ShapeDtypeStruct((N, 16), jnp.float32),
    )(p, b1.reshape(1, 64), W2)

    q = _get_spmm(16)(support2, src, dst, ew)[:, :N]

    logits, logp = pl.pallas_call(
        _stage3_body,
        out_shape=[jax.ShapeDtypeStruct((N, 16), jnp.float32),
                   jax.ShapeDtypeStruct((N, 16), jnp.float32)],
    )(q, b2.reshape(1, 16))

    return (logits, logp)


# speedup vs baseline: 15.3116x; 1.5844x over previous
import functools

import jax
import jax.numpy as jnp
from jax import lax
from jax.experimental import pallas as pl
from jax.experimental.pallas import tpu as pltpu
from jax.experimental.pallas import tpu_sc as plsc

N = 10000
E = 320000
CH = 128
NCORE = 2
NSUB = 16
NWORK = NCORE * NSUB
ROWS_PER_SUB = 640
ZROWS = 128
NPAD = ROWS_PER_SUB * NSUB
NCHUNK = E // CH


CHW_BASE = NCHUNK // NWORK
CHW_EXTRA = NCHUNK - CHW_BASE * NWORK
EBASE = CHW_BASE * CH
EBUF = (CHW_BASE + 1) * CH


def _make_spmm(D, TN):
    mesh = plsc.VectorSubcoreMesh(core_axis_name="c", subcore_axis_name="s",
                                  num_cores=NCORE, num_subcores=NSUB)
    nvec = D // 16

    @functools.partial(
        pl.kernel,
        out_type=jax.ShapeDtypeStruct((NCORE, NPAD, D), jnp.float32),
        mesh=mesh,
        scratch_types=[
            pltpu.VMEM((EBUF,), jnp.int32),
            pltpu.VMEM((EBUF,), jnp.int32),
            pltpu.VMEM((EBUF,), jnp.float32),
            pltpu.VMEM((4, CH, D), jnp.float32),
            pltpu.VMEM((ZROWS, D), jnp.float32),
            pltpu.VMEM_SHARED((NPAD, D), jnp.float32),
            pltpu.SemaphoreType.DMA,
            pltpu.SemaphoreType.DMA,
        ],
        compiler_params=pltpu.CompilerParams(use_tc_tiling_on_sc=False),
    )
    def spmm(table_ref, src_ref, dst_ref, w_ref, out_ref,
             sidx, didx, wbuf, rows, zbuf, acc, gsem, ssem):
        c = lax.axis_index("c")
        s = lax.axis_index("s")
        wid = c * NSUB + s
        nch = CHW_BASE + jnp.where(wid < CHW_EXTRA, 1, 0)
        e0 = (wid * CHW_BASE + jnp.minimum(wid, CHW_EXTRA)) * CH

        pre1 = pltpu.async_copy(src_ref.at[pl.ds(e0, EBASE)],
                                sidx.at[pl.ds(0, EBASE)], gsem)
        pre2 = pltpu.async_copy(dst_ref.at[pl.ds(e0, EBASE)],
                                didx.at[pl.ds(0, EBASE)], gsem)
        pre3 = pltpu.async_copy(w_ref.at[pl.ds(e0, EBASE)],
                                wbuf.at[pl.ds(0, EBASE)], gsem)

        zv = jnp.zeros((16,), jnp.float32)

        def zbody(i, carry):
            for jj in range(nvec):
                zbuf[i, pl.ds(jj * 16, 16)] = zv
            return carry

        lax.fori_loop(0, ZROWS, zbody, 0)
        for k in range(ROWS_PER_SUB // ZROWS):
            pltpu.sync_copy(zbuf,
                            acc.at[pl.ds(s * ROWS_PER_SUB + k * ZROWS, ZROWS)])

        @pl.when(wid < CHW_EXTRA)
        def _():
            pltpu.sync_copy(src_ref.at[pl.ds(e0 + EBASE, CH)],
                            sidx.at[pl.ds(EBASE, CH)])
            pltpu.sync_copy(dst_ref.at[pl.ds(e0 + EBASE, CH)],
                            didx.at[pl.ds(EBASE, CH)])
            pltpu.sync_copy(w_ref.at[pl.ds(e0 + EBASE, CH)],
                            wbuf.at[pl.ds(EBASE, CH)])

        pre1.wait()
        pre2.wait()
        pre3.wait()
        plsc.subcore_barrier()

        pltpu.async_copy(table_ref.at[sidx.at[pl.ds(0, CH)]], rows.at[0],
                         gsem)

        @pl.when(1 < nch)
        def _():
            pltpu.async_copy(table_ref.at[sidx.at[pl.ds(CH, CH)]],
                             rows.at[1], gsem)

        def chunk_body(t, carry):
            b = lax.rem(t, 4)
            pltpu.make_async_copy(table_ref.at[sidx.at[pl.ds(0, CH)]],
                                  rows.at[b], gsem).wait()

            @pl.when(t >= 2)
            def _():
                pltpu.make_async_copy(table_ref.at[sidx.at[pl.ds(0, CH)]],
                                      rows.at[b], ssem).wait()

            @pl.when(t + 2 < nch)
            def _():
                pltpu.async_copy(
                    table_ref.at[sidx.at[pl.ds((t + 2) * CH, CH)]],
                    rows.at[lax.rem(t + 2, 4)], gsem)

            @plsc.parallel_loop(0, CH // 16, unroll=2)
            def sbody(g):
                wvec = wbuf[pl.ds(t * CH + g * 16, 16)]
                for k in range(16):
                    wv = wvec[k]
                    i = g * 16 + k
                    for jj in range(nvec):
                        sl = pl.ds(jj * 16, 16)
                        rows[b, i, sl] = rows[b, i, sl] * wv
            pltpu.async_copy(rows.at[b], acc.at[didx.at[pl.ds(t * CH, CH)]],
                             ssem, add=True)
            return carry

        lax.fori_loop(0, nch, chunk_body, 0)
        pltpu.make_async_copy(table_ref.at[sidx.at[pl.ds(0, CH)]],
                              rows.at[0], ssem).wait()
        pltpu.make_async_copy(table_ref.at[sidx.at[pl.ds(0, CH)]],
                              rows.at[0], ssem).wait()
        plsc.subcore_barrier()
        pltpu.sync_copy(acc.at[pl.ds(s * ROWS_PER_SUB, ROWS_PER_SUB)],
                        out_ref.at[c, pl.ds(s * ROWS_PER_SUB, ROWS_PER_SUB)])

    return spmm


_SPMM_CACHE = {}


def _get_spmm(D, TN):
    if (D, TN) not in _SPMM_CACHE:
        _SPMM_CACHE[(D, TN)] = _make_spmm(D, TN)
    return _SPMM_CACHE[(D, TN)]


def _mm1_body(x_ref, w_ref, o_ref):
    o_ref[...] = jnp.dot(x_ref[...], w_ref[...],
                         preferred_element_type=jnp.float32)


def _stage2_body(p_ref, b1_ref, w2_ref, o_ref):
    h = jnp.maximum(p_ref[0] + p_ref[1] + b1_ref[...], 0.0)
    o_ref[...] = jnp.dot(h, w2_ref[...], preferred_element_type=jnp.float32)


def _stage3_body(q_ref, b2_ref, lo_ref, lp_ref):
    lg = q_ref[0, :N] + q_ref[1, :N] + b2_ref[...]
    m = jnp.max(lg, axis=1, keepdims=True)
    se = jnp.sum(jnp.exp(lg - m), axis=1, keepdims=True)
    lo_ref[...] = lg
    lp_ref[...] = lg - m - jnp.log(se)


def kernel(x, edge_index, edge_weight, W1, b1, W2, b2):
    src = edge_index[1].astype(jnp.int32)
    dst = edge_index[0].astype(jnp.int32)
    ew = edge_weight.astype(jnp.float32)

    support1 = pl.pallas_call(
        _mm1_body,
        out_shape=jax.ShapeDtypeStruct((N, 64), jnp.float32),
    )(x, W1)

    p = _get_spmm(64, N)(support1, src, dst, ew)

    support2 = pl.pallas_call(
        _stage2_body,
        out_shape=jax.ShapeDtypeStruct((NPAD, 16), jnp.float32),
    )(p, b1.reshape(1, 64), W2)

    q = _get_spmm(16, NPAD)(support2, src, dst, ew)

    logits, logp = pl.pallas_call(
        _stage3_body,
        out_shape=[jax.ShapeDtypeStruct((N, 16), jnp.float32),
                   jax.ShapeDtypeStruct((N, 16), jnp.float32)],
    )(q, b2.reshape(1, 16))

    return (logits, logp)
